# baseline (speedup 1.0000x reference)
import jax
import jax.numpy as jnp
import numpy as np
from jax.experimental import pallas as pl
from jax.experimental.pallas import tpu as pltpu

B = 4096
DIM = 256
K = 8192
LEVELS = 3
BETA = 0.25
EPS = 1e-05
USAGE_REG = 0.001

BT = 256
NB = B // BT
DIST_PREC = jax.lax.Precision.DEFAULT
GATHER_PREC = jax.lax.Precision.DEFAULT


def _rvq_body(x_ref, w_ref, out_ref, codes_ref, commit_ref, usage_ref,
              res_ref, pacc_ref, e2_ref, wcat_ref):
    lvl = pl.program_id(0)
    i = pl.program_id(1)
    rows = pl.ds(i * BT, BT)

    W = w_ref[0]

    @pl.when(lvl == 0)
    def _init_res():
        res_ref[rows, :] = x_ref[...]

    @pl.when((lvl == 0) & (i == 0))
    def _init_scalars():
        commit_ref[...] = jnp.zeros((1, 1), jnp.float32)

    @pl.when(i == 0)
    def _split_codebook():
        hi = W.astype(jnp.bfloat16)
        r1 = W - hi.astype(jnp.float32)
        mid = r1.astype(jnp.bfloat16)
        lo = (r1 - mid.astype(jnp.float32)).astype(jnp.bfloat16)
        wcat_ref[:, 0:DIM] = hi
        wcat_ref[:, DIM:2 * DIM] = mid
        wcat_ref[:, 2 * DIM:3 * DIM] = lo
        e2_ref[...] = jnp.sum(W * W, axis=1)[None, :]

    res = res_ref[rows, :]
    x2 = jnp.sum(res * res, axis=1, keepdims=True)
    xe = jax.lax.dot_general(res, W, (((1,), (1,)), ((), ())),
                             precision=DIST_PREC,
                             preferred_element_type=jnp.float32)
    d = x2 - 2.0 * xe + e2_ref[...]

    iota = jax.lax.broadcasted_iota(jnp.int32, (BT, K), 1)
    dmin = jnp.min(d, axis=1, keepdims=True)
    idx = jnp.min(jnp.where(d == dmin, iota, K), axis=1)
    codes_ref[...] = idx.reshape(1, 1, BT)

    p = jnp.exp(dmin - d)
    s = jnp.sum(p, axis=1, keepdims=True)
    psum = jnp.sum(p / s, axis=0)[None, :]

    @pl.when(i == 0)
    def _init_pacc():
        pacc_ref[...] = psum

    @pl.when(i != 0)
    def _acc_pacc():
        pacc_ref[...] = pacc_ref[...] + psum

    onehot = (iota == idx[:, None]).astype(jnp.bfloat16)
    qcat = jax.lax.dot_general(onehot, wcat_ref[...],
                               (((1,), (0,)), ((), ())),
                               precision=GATHER_PREC,
                               preferred_element_type=jnp.float32)
    q = (qcat[:, 0:DIM] + qcat[:, DIM:2 * DIM]) + qcat[:, 2 * DIM:3 * DIM]

    res_new = res - q
    res_ref[rows, :] = res_new

    c = (BETA / (LEVELS * B * DIM)) * jnp.sum(res_new * res_new)
    commit_ref[...] = commit_ref[...] + c.reshape(1, 1)

    @pl.when(i == NB - 1)
    def _finish_level():
        avg = jnp.clip(pacc_ref[...] * (1.0 / B), EPS, None)
        ent = -jnp.sum(avg * jnp.log(avg))
        kl = jnp.float32(np.log(K)) - ent
        v = (USAGE_REG / LEVELS) * kl

        @pl.when(lvl == 0)
        def _u0():
            usage_ref[...] = v.reshape(1, 1)

        @pl.when(lvl != 0)
        def _un():
            usage_ref[...] = usage_ref[...] + v.reshape(1, 1)

    @pl.when(lvl == LEVELS - 1)
    def _write_out():
        out_ref[...] = x_ref[...] - res_new


def kernel(x, codebooks):
    out, codes, commit, usage = pl.pallas_call(
        _rvq_body,
        grid=(LEVELS, NB),
        in_specs=[
            pl.BlockSpec((BT, DIM), lambda l, i: (i, 0)),
            pl.BlockSpec((1, K, DIM), lambda l, i: (l, 0, 0)),
        ],
        out_specs=[
            pl.BlockSpec((BT, DIM), lambda l, i: (i, 0)),
            pl.BlockSpec((1, 1, BT), lambda l, i: (l, 0, i)),
            pl.BlockSpec((1, 1), lambda l, i: (0, 0)),
            pl.BlockSpec((1, 1), lambda l, i: (0, 0)),
        ],
        out_shape=[
            jax.ShapeDtypeStruct((B, DIM), jnp.float32),
            jax.ShapeDtypeStruct((LEVELS, 1, B), jnp.int32),
            jax.ShapeDtypeStruct((1, 1), jnp.float32),
            jax.ShapeDtypeStruct((1, 1), jnp.float32),
        ],
        compiler_params=pltpu.CompilerParams(
            vmem_limit_bytes=63 * 1024 * 1024,
        ),
        scratch_shapes=[
            pltpu.VMEM((B, DIM), jnp.float32),
            pltpu.VMEM((1, K), jnp.float32),
            pltpu.VMEM((1, K), jnp.float32),
            pltpu.VMEM((K, 3 * DIM), jnp.bfloat16),
        ],
    )(x, codebooks)
    codes_bl = codes.reshape(LEVELS, B).T
    return out, codes_bl, commit[0, 0], usage[0, 0]


# speedup vs baseline: 1.0654x; 1.0654x over previous
import jax
import jax.numpy as jnp
import numpy as np
from jax.experimental import pallas as pl
from jax.experimental.pallas import tpu as pltpu

B = 4096
DIM = 256
K = 8192
LEVELS = 3
BETA = 0.25
EPS = 1e-05
USAGE_REG = 0.001

BT = 256
NB = B // BT
DIST_PREC = jax.lax.Precision.DEFAULT
GATHER_PREC = jax.lax.Precision.DEFAULT


def _rvq_body(x_ref, w_ref, out_ref, codes_ref, commit_ref, usage_ref,
              res_ref, pacc_ref, wcat_ref):
    lvl = pl.program_id(0)
    i = pl.program_id(1)
    rows = pl.ds(i * BT, BT)

    W = w_ref[0]

    @pl.when(lvl == 0)
    def _init_res():
        res_ref[rows, :] = x_ref[...]

    @pl.when((lvl == 0) & (i == 0))
    def _init_scalars():
        commit_ref[...] = jnp.zeros((1, 1), jnp.float32)

    @pl.when(i == 0)
    def _split_codebook():
        hi = W.astype(jnp.bfloat16)
        r1 = W - hi.astype(jnp.float32)
        mid = r1.astype(jnp.bfloat16)
        lo = (r1 - mid.astype(jnp.float32)).astype(jnp.bfloat16)
        wcat_ref[:, 0:DIM] = hi
        wcat_ref[:, DIM:2 * DIM] = mid
        wcat_ref[:, 2 * DIM:3 * DIM] = lo

    res = res_ref[rows, :]
    x2 = jnp.sum(res * res, axis=1, keepdims=True)
    e2 = jnp.sum(W * W, axis=1)
    xe = jax.lax.dot_general(res, W, (((1,), (1,)), ((), ())),
                             precision=DIST_PREC,
                             preferred_element_type=jnp.float32)
    d = x2 - 2.0 * xe + e2[None, :]

    iota = jax.lax.broadcasted_iota(jnp.int32, (BT, K), 1)
    dmin = jnp.min(d, axis=1, keepdims=True)
    idx = jnp.argmin(d, axis=1).astype(jnp.int32)
    codes_ref[...] = idx.reshape(1, 1, BT)

    p = jnp.exp(dmin - d)
    s = jnp.sum(p, axis=1, keepdims=True)
    psum = jnp.sum(p / s, axis=0)[None, :]

    @pl.when(i == 0)
    def _init_pacc():
        pacc_ref[...] = psum

    @pl.when(i != 0)
    def _acc_pacc():
        pacc_ref[...] = pacc_ref[...] + psum

    onehot = (iota == idx[:, None]).astype(jnp.bfloat16)
    qcat = jax.lax.dot_general(onehot, wcat_ref[...],
                               (((1,), (0,)), ((), ())),
                               precision=GATHER_PREC,
                               preferred_element_type=jnp.float32)
    q = (qcat[:, 0:DIM] + qcat[:, DIM:2 * DIM]) + qcat[:, 2 * DIM:3 * DIM]

    res_new = res - q
    res_ref[rows, :] = res_new

    c = (BETA / (LEVELS * B * DIM)) * jnp.sum(res_new * res_new)
    commit_ref[...] = commit_ref[...] + c.reshape(1, 1)

    @pl.when(i == NB - 1)
    def _finish_level():
        avg = jnp.clip(pacc_ref[...] * (1.0 / B), EPS, None)
        ent = -jnp.sum(avg * jnp.log(avg))
        kl = jnp.float32(np.log(K)) - ent
        v = (USAGE_REG / LEVELS) * kl

        @pl.when(lvl == 0)
        def _u0():
            usage_ref[...] = v.reshape(1, 1)

        @pl.when(lvl != 0)
        def _un():
            usage_ref[...] = usage_ref[...] + v.reshape(1, 1)

    @pl.when(lvl == LEVELS - 1)
    def _write_out():
        out_ref[...] = x_ref[...] - res_new


def kernel(x, codebooks):
    out, codes, commit, usage = pl.pallas_call(
        _rvq_body,
        grid=(LEVELS, NB),
        in_specs=[
            pl.BlockSpec((BT, DIM), lambda l, i: (i, 0)),
            pl.BlockSpec((1, K, DIM), lambda l, i: (l, 0, 0)),
        ],
        out_specs=[
            pl.BlockSpec((BT, DIM), lambda l, i: (i, 0)),
            pl.BlockSpec((1, 1, BT), lambda l, i: (l, 0, i)),
            pl.BlockSpec((1, 1), lambda l, i: (0, 0)),
            pl.BlockSpec((1, 1), lambda l, i: (0, 0)),
        ],
        out_shape=[
            jax.ShapeDtypeStruct((B, DIM), jnp.float32),
            jax.ShapeDtypeStruct((LEVELS, 1, B), jnp.int32),
            jax.ShapeDtypeStruct((1, 1), jnp.float32),
            jax.ShapeDtypeStruct((1, 1), jnp.float32),
        ],
        compiler_params=pltpu.CompilerParams(
            vmem_limit_bytes=63 * 1024 * 1024,
        ),
        scratch_shapes=[
            pltpu.VMEM((B, DIM), jnp.float32),
            pltpu.VMEM((1, K), jnp.float32),
            pltpu.VMEM((K, 3 * DIM), jnp.bfloat16),
        ],
    )(x, codebooks)
    codes_bl = codes.reshape(LEVELS, B).T
    return out, codes_bl, commit[0, 0], usage[0, 0]


# speedup vs baseline: 1.1208x; 1.0520x over previous
import jax
import jax.numpy as jnp
import numpy as np
from jax.experimental import pallas as pl
from jax.experimental.pallas import tpu as pltpu

B = 4096
DIM = 256
K = 8192
LEVELS = 3
BETA = 0.25
EPS = 1e-05
USAGE_REG = 0.001

BT = 256
NB = B // BT
DIST_PREC = jax.lax.Precision.DEFAULT
GATHER_PREC = jax.lax.Precision.DEFAULT


def _rvq_body(x_ref, w_ref, out_ref, codes_ref, commit_ref, usage_ref,
              res_ref, pacc_ref, e2_ref, wcat_ref):
    lvl = pl.program_id(0)
    i = pl.program_id(1)
    rows = pl.ds(i * BT, BT)

    W = w_ref[0]

    @pl.when(lvl == 0)
    def _init_res():
        res_ref[rows, :] = x_ref[...]

    @pl.when((lvl == 0) & (i == 0))
    def _init_scalars():
        commit_ref[...] = jnp.zeros((1, 1), jnp.float32)

    @pl.when(i == 0)
    def _split_codebook():
        hi = W.astype(jnp.bfloat16)
        r1 = W - hi.astype(jnp.float32)
        mid = r1.astype(jnp.bfloat16)
        lo = (r1 - mid.astype(jnp.float32)).astype(jnp.bfloat16)
        wcat_ref[:, 0:DIM] = hi
        wcat_ref[:, DIM:2 * DIM] = mid
        wcat_ref[:, 2 * DIM:3 * DIM] = lo
        e2_ref[...] = jnp.broadcast_to(
            jnp.sum(W * W, axis=1)[None, :], (8, K))

    res = res_ref[rows, :]
    x2 = jnp.sum(res * res, axis=1, keepdims=True)
    xe = jax.lax.dot_general(res, W, (((1,), (1,)), ((), ())),
                             precision=DIST_PREC,
                             preferred_element_type=jnp.float32)
    t = x2 - 2.0 * xe
    d = (t.reshape(BT // 8, 8, K) + e2_ref[...][None, :, :]).reshape(BT, K)

    iota = jax.lax.broadcasted_iota(jnp.int32, (BT, K), 1)
    dmin = jnp.min(d, axis=1, keepdims=True)
    idx = jnp.argmin(d, axis=1).astype(jnp.int32)
    codes_ref[...] = idx.reshape(1, 1, BT)

    p = jnp.exp(dmin - d)
    s = jnp.sum(p, axis=1, keepdims=True)
    psum = jnp.sum(p / s, axis=0)[None, :]

    @pl.when(i == 0)
    def _init_pacc():
        pacc_ref[...] = psum

    @pl.when(i != 0)
    def _acc_pacc():
        pacc_ref[...] = pacc_ref[...] + psum

    onehot = (iota == idx[:, None]).astype(jnp.bfloat16)
    qcat = jax.lax.dot_general(onehot, wcat_ref[...],
                               (((1,), (0,)), ((), ())),
                               precision=GATHER_PREC,
                               preferred_element_type=jnp.float32)
    q = (qcat[:, 0:DIM] + qcat[:, DIM:2 * DIM]) + qcat[:, 2 * DIM:3 * DIM]

    res_new = res - q
    res_ref[rows, :] = res_new

    c = (BETA / (LEVELS * B * DIM)) * jnp.sum(res_new * res_new)
    commit_ref[...] = commit_ref[...] + c.reshape(1, 1)

    @pl.when(i == NB - 1)
    def _finish_level():
        avg = jnp.clip(pacc_ref[...] * (1.0 / B), EPS, None)
        ent = -jnp.sum(avg * jnp.log(avg))
        kl = jnp.float32(np.log(K)) - ent
        v = (USAGE_REG / LEVELS) * kl

        @pl.when(lvl == 0)
        def _u0():
            usage_ref[...] = v.reshape(1, 1)

        @pl.when(lvl != 0)
        def _un():
            usage_ref[...] = usage_ref[...] + v.reshape(1, 1)

    @pl.when(lvl == LEVELS - 1)
    def _write_out():
        out_ref[...] = x_ref[...] - res_new


def kernel(x, codebooks):
    out, codes, commit, usage = pl.pallas_call(
        _rvq_body,
        grid=(LEVELS, NB),
        in_specs=[
            pl.BlockSpec((BT, DIM), lambda l, i: (i, 0)),
            pl.BlockSpec((1, K, DIM), lambda l, i: (l, 0, 0)),
        ],
        out_specs=[
            pl.BlockSpec((BT, DIM), lambda l, i: (i, 0)),
            pl.BlockSpec((1, 1, BT), lambda l, i: (l, 0, i)),
            pl.BlockSpec((1, 1), lambda l, i: (0, 0)),
            pl.BlockSpec((1, 1), lambda l, i: (0, 0)),
        ],
        out_shape=[
            jax.ShapeDtypeStruct((B, DIM), jnp.float32),
            jax.ShapeDtypeStruct((LEVELS, 1, B), jnp.int32),
            jax.ShapeDtypeStruct((1, 1), jnp.float32),
            jax.ShapeDtypeStruct((1, 1), jnp.float32),
        ],
        compiler_params=pltpu.CompilerParams(
            vmem_limit_bytes=63 * 1024 * 1024,
        ),
        scratch_shapes=[
            pltpu.VMEM((B, DIM), jnp.float32),
            pltpu.VMEM((1, K), jnp.float32),
            pltpu.VMEM((8, K), jnp.float32),
            pltpu.VMEM((K, 3 * DIM), jnp.bfloat16),
        ],
    )(x, codebooks)
    codes_bl = codes.reshape(LEVELS, B).T
    return out, codes_bl, commit[0, 0], usage[0, 0]
